# baseline (speedup 1.0000x reference)
import functools
import math

import jax
import jax.numpy as jnp
from jax.experimental import pallas as pl

_N_POS = 1000000
_D = 64
_NB = math.ceil(math.log2(_N_POS))
_PACK = 4
_LANE = 32
_ROWS_PER_BLK = 512


def _tc_body(x4_ref, sel_ref, tbig_ref, out_ref):
    x4 = x4_ref[...].astype(jnp.float32)
    xf = jax.lax.dot(x4, sel_ref[...],
                     precision=jax.lax.Precision.HIGHEST)
    xi = xf.astype(jnp.int32)
    sh = jax.lax.broadcasted_iota(jnp.int32, xi.shape, 1) & (_LANE - 1)
    bits = ((xi >> sh) & 1).astype(jnp.float32)
    out_ref[...] = jax.lax.dot(bits, tbig_ref[...],
                               precision=jax.lax.Precision.HIGHEST)


def _tc_embed(x_flat, table, interpret=False):
    t = x_flat.shape[0]
    t4 = t // _PACK
    x4 = x_flat.reshape(t4, _PACK)

    jj = jnp.arange(_PACK * _LANE, dtype=jnp.int32) // _LANE
    sel = (jj[None, :] == jnp.arange(_PACK, dtype=jnp.int32)[:, None])
    sel = sel.astype(jnp.float32)

    tpad = jnp.zeros((_LANE, _D), jnp.float32).at[:_NB].set(table[:_NB])
    tbig = jnp.zeros((_PACK, _LANE, _PACK, _D), jnp.float32)
    tbig = tbig.at[jnp.arange(_PACK), :, jnp.arange(_PACK), :].set(tpad)
    tbig = tbig.reshape(_PACK * _LANE, _PACK * _D)

    grid = t4 // _ROWS_PER_BLK
    out4 = pl.pallas_call(
        _tc_body,
        grid=(grid,),
        in_specs=[
            pl.BlockSpec((_ROWS_PER_BLK, _PACK), lambda i: (i, 0)),
            pl.BlockSpec((_PACK, _PACK * _LANE), lambda i: (0, 0)),
            pl.BlockSpec((_PACK * _LANE, _PACK * _D), lambda i: (0, 0)),
        ],
        out_specs=pl.BlockSpec((_ROWS_PER_BLK, _PACK * _D), lambda i: (i, 0)),
        out_shape=jax.ShapeDtypeStruct((t4, _PACK * _D), jnp.float32),
        interpret=interpret,
    )(x4, sel, tbig)
    return out4.reshape(t, _D)


def kernel(x, table):
    x_flat = x.reshape(-1)
    return _tc_embed(x_flat, table)


# speedup vs baseline: 1.8864x; 1.8864x over previous
import functools
import math

import jax
import jax.numpy as jnp
from jax import lax
from jax.experimental import pallas as pl
from jax.experimental.pallas import tpu as pltpu
from jax.experimental.pallas import tpu_sc as plsc

_N_POS = 1000000
_D = 64
_NB = math.ceil(math.log2(_N_POS))
_LO = 10
_HI = _NB - _LO
_T2 = (1 << _LO) + (1 << _HI)

_NC = 2
_NS = 16
_NW = _NC * _NS
_L = 16

_CHUNK = 128


def _t2_body(tlo_ref, thi_ref, out_ref):
    n = 1 << _LO
    v = lax.broadcasted_iota(jnp.int32, (n, 32), 0)
    b = lax.broadcasted_iota(jnp.int32, (n, 32), 1)
    bits = ((v >> b) & 1).astype(jnp.float32)
    out_ref[:n] = lax.dot(bits, tlo_ref[...],
                          precision=lax.Precision.HIGHEST)
    out_ref[n:] = lax.dot(bits, thi_ref[...],
                          precision=lax.Precision.HIGHEST)


def _build_table2(table, interpret=False):
    tlo = jnp.zeros((32, _D), jnp.float32).at[:_LO].set(table[:_LO])
    thi = jnp.zeros((32, _D), jnp.float32).at[:_HI].set(table[_LO:_NB])
    return pl.pallas_call(
        _t2_body,
        out_shape=jax.ShapeDtypeStruct((_T2, _D), jnp.float32),
        interpret=interpret,
    )(tlo, thi)


def _sc_embed(x_flat, t2):
    t = x_flat.shape[0]
    per_w = t // _NW
    n_pairs = per_w // (2 * _CHUNK)
    mesh = plsc.VectorSubcoreMesh(core_axis_name="c", subcore_axis_name="s")

    nbuf = 4
    n_steps = per_w // (nbuf * _CHUNK)
    idx_types = []
    for _i in range(nbuf):
        idx_types += [pltpu.VMEM((_CHUNK,), jnp.int32),
                      pltpu.VMEM((_CHUNK,), jnp.int32)]
    buf_types = [pltpu.VMEM((_CHUNK, _D), jnp.float32) for _i in range(nbuf)]

    @functools.partial(
        pl.kernel, mesh=mesh,
        out_type=jax.ShapeDtypeStruct((t, _D), jnp.float32),
        scratch_types=(
            [pltpu.VMEM((per_w,), jnp.int32)] + idx_types + buf_types
            + [pltpu.VMEM_SHARED((_T2, _D), jnp.float32),
               pltpu.SemaphoreType.DMA,
               pltpu.SemaphoreType.DMA]
        ),
    )
    def k(x_hbm, t2_hbm, out_hbm, x_v, *rest):
        idx_refs = rest[:2 * nbuf]
        bufs = rest[2 * nbuf:3 * nbuf]
        t2_spm, gs, ws = rest[3 * nbuf:]
        wid = lax.axis_index("s") * _NC + lax.axis_index("c")
        sid = lax.axis_index("s")
        base = wid * per_w
        @pl.when(sid == 0)
        def _():
            pltpu.sync_copy(t2_hbm, t2_spm)
        plsc.subcore_barrier()
        pltpu.sync_copy(x_hbm.at[pl.ds(base, per_w)], x_v)

        def compute_idx(off, ilo, ihi):
            for i in range(_CHUNK // _L):
                v = x_v[pl.ds(off + i * _L, _L)]
                ilo[pl.ds(i * _L, _L)] = v & ((1 << _LO) - 1)
                ihi[pl.ds(i * _L, _L)] = (v >> _LO) + (1 << _LO)

        def step(b, _):
            j0 = b * 2
            off0 = j0 * _CHUNK
            off1 = (j0 + 1) * _CHUNK
            compute_idx(off0, idx_refs[0], idx_refs[1])
            pltpu.async_copy(t2_spm.at[idx_refs[0]], bufs[0], gs).wait()
            pltpu.async_copy(t2_spm.at[idx_refs[1]], bufs[0], gs, add=True).wait()
            w0 = pltpu.async_copy(
                bufs[0], out_hbm.at[pl.ds(base + off0, _CHUNK), :], ws)
            compute_idx(off1, idx_refs[2], idx_refs[3])
            pltpu.async_copy(t2_spm.at[idx_refs[2]], bufs[1], gs).wait()
            pltpu.async_copy(t2_spm.at[idx_refs[3]], bufs[1], gs, add=True).wait()
            w0.wait()
            pltpu.async_copy(
                bufs[1], out_hbm.at[pl.ds(base + off1, _CHUNK), :], ws).wait()
            return 0

        lax.fori_loop(0, per_w // (2 * _CHUNK), step, 0)

    return k(x_flat, t2)


def kernel(x, table):
    x_flat = x.reshape(-1)
    t2 = _build_table2(table)
    return _sc_embed(x_flat, t2)
